# Initial kernel scaffold; baseline (speedup 1.0000x reference)
import jax
import jax.numpy as jnp
from jax.experimental import pallas as pl


def kernel(input_ids, weights):
    raise NotImplementedError("write your pallas kernel here")


# speedup vs baseline: 2.7552x; 2.7552x over previous
import jax
import jax.numpy as jnp
from jax.experimental import pallas as pl


def _copy_block(w_ref, o_ref):
    o_ref[...] = w_ref[...]


def kernel(input_ids, weights):
    seq_len = input_ids.shape[-1]
    dim = weights.shape[1]
    blk = 512
    assert seq_len % blk == 0
    return pl.pallas_call(
        _copy_block,
        grid=(seq_len // blk,),
        in_specs=[pl.BlockSpec((blk, dim), lambda i: (i, 0))],
        out_specs=pl.BlockSpec((blk, dim), lambda i: (i, 0)),
        out_shape=jax.ShapeDtypeStruct((seq_len, dim), weights.dtype),
    )(weights)


# speedup vs baseline: 3.4165x; 1.2400x over previous
import jax
import jax.numpy as jnp
from jax.experimental import pallas as pl


def _copy_block(w_ref, o_ref):
    o_ref[...] = w_ref[...]


def kernel(input_ids, weights):
    seq_len = input_ids.shape[-1]
    dim = weights.shape[1]
    blk = 2048
    assert seq_len % blk == 0
    return pl.pallas_call(
        _copy_block,
        grid=(seq_len // blk,),
        in_specs=[pl.BlockSpec((blk, dim), lambda i: (i, 0))],
        out_specs=pl.BlockSpec((blk, dim), lambda i: (i, 0)),
        out_shape=jax.ShapeDtypeStruct((seq_len, dim), weights.dtype),
    )(weights)
